# baseline (speedup 1.0000x reference)
import functools

import jax
import jax.numpy as jnp
from jax import lax
from jax.experimental import pallas as pl
from jax.experimental.pallas import tpu as pltpu
from jax.experimental.pallas import tpu_sc as plsc

_NUM_EXPERTS = 64
_TOP_K = 2
_ALPHA = 0.01
_TILE = 4096

_NC = 2
_NS = 16
_NW = _NC * _NS
_L = 16


def _matmul_body(x_ref, wt_ref, lt_out):
    tile = x_ref.shape[0]
    logits = jnp.dot(x_ref[...], wt_ref[...],
                     preferred_element_type=jnp.float32)
    sub = tile // 1024
    lt = logits.reshape(sub, 1024, _NUM_EXPERTS).transpose(0, 2, 1)
    lt_out[...] = lt


def _sc_router_body(lt_hbm, w_hbm, e_hbm, ps_hbm, cnt_hbm,
                    blk, wstage, estage, psacc, cntacc, dsem):
    wid = lax.axis_index("s") * _NC + lax.axis_index("c")
    chunk = blk.shape[1]
    n_groups = chunk // _L

    cp = pltpu.async_copy(lt_hbm.at[wid], blk, dsem)

    zf = jnp.zeros((_L,), jnp.float32)

    def zero_body(e, _):
        psacc[e, :] = zf
        cntacc[e, :] = zf
        return 0

    lax.fori_loop(0, _NUM_EXPERTS, zero_body, 0)
    cp.wait()

    def group_body(g, _):
        sl = pl.ds(g * _L, _L)

        def top2_body(e, carry):
            m1, i1, m2, i2 = carry
            v = blk[e, sl]
            ev = jnp.full((_L,), e, jnp.int32)
            gt1 = v > m1
            gt2 = v > m2
            m2n = jnp.where(gt1, m1, jnp.where(gt2, v, m2))
            i2n = jnp.where(gt1, i1, jnp.where(gt2, ev, i2))
            m1n = jnp.where(gt1, v, m1)
            i1n = jnp.where(gt1, ev, i1)
            return (m1n, i1n, m2n, i2n)

        init = (blk[0, sl], jnp.zeros((_L,), jnp.int32),
                jnp.full((_L,), -jnp.inf, jnp.float32),
                jnp.zeros((_L,), jnp.int32))
        m1, i1, m2, i2 = lax.fori_loop(1, _NUM_EXPERTS, top2_body, init)

        def z_body(e, z):
            ev = jnp.exp(blk[e, sl] - m1)
            blk[e, sl] = ev
            return z + ev

        z = lax.fori_loop(0, _NUM_EXPERTS, z_body, jnp.zeros((_L,),
                                                             jnp.float32))
        rz = 1.0 / z

        one = jnp.ones((_L,), jnp.float32)

        def ps_body(e, _):
            psacc[e, :] = psacc[e, :] + blk[e, sl] * rz
            hits = (jnp.where(i1 == e, one, 0.0) +
                    jnp.where(i2 == e, one, 0.0))
            cntacc[e, :] = cntacc[e, :] + hits
            return 0

        lax.fori_loop(0, _NUM_EXPERTS, ps_body, 0)

        t = jnp.exp(m2 - m1)
        w1 = 1.0 / (1.0 + t)
        wstage[0, sl] = w1
        wstage[1, sl] = 1.0 - w1
        estage[0, sl] = i1
        estage[1, sl] = i2
        return 0

    lax.fori_loop(0, n_groups, group_body, 0)

    base = wid * chunk
    pltpu.sync_copy(wstage.at[0], w_hbm.at[0, pl.ds(base, chunk)])
    pltpu.sync_copy(wstage.at[1], w_hbm.at[1, pl.ds(base, chunk)])
    pltpu.sync_copy(estage.at[0], e_hbm.at[0, pl.ds(base, chunk)])
    pltpu.sync_copy(estage.at[1], e_hbm.at[1, pl.ds(base, chunk)])
    pltpu.sync_copy(psacc, ps_hbm.at[wid])
    pltpu.sync_copy(cntacc, cnt_hbm.at[wid])


def _finish_body(w2_ref, e2_ref, ps_ref, cnt_ref, w_out, e_out, loss_out, *,
                 num_tokens):
    w_out[...] = w2_ref[...].T
    e_out[...] = e2_ref[...].T

    @pl.when(pl.program_id(0) == 0)
    def _loss():
        psum = jnp.sum(ps_ref[...], axis=(0, 2))
        cnt = jnp.sum(cnt_ref[...], axis=(0, 2))
        scale = _ALPHA * _NUM_EXPERTS / (num_tokens * num_tokens)
        loss_out[...] = scale * jnp.sum(psum * cnt, keepdims=True)[None]


def kernel(hidden_states, W_gate):
    batch, seq, hidden = hidden_states.shape
    num_tokens = batch * seq
    x = hidden_states.reshape(num_tokens, hidden)
    wt = W_gate.T
    chunk = num_tokens // _NW
    n_steps = num_tokens // _TILE
    sub = _TILE // 1024

    logits_t = pl.pallas_call(
        _matmul_body,
        grid=(n_steps,),
        in_specs=[
            pl.BlockSpec((_TILE, hidden), lambda i: (i, 0)),
            pl.BlockSpec((hidden, _NUM_EXPERTS), lambda i: (0, 0)),
        ],
        out_specs=pl.BlockSpec((sub, _NUM_EXPERTS, 1024), lambda i: (i, 0, 0)),
        out_shape=jax.ShapeDtypeStruct((num_tokens // 1024, _NUM_EXPERTS,
                                        1024), jnp.float32),
    )(x, wt)

    sc_router = pl.kernel(
        _sc_router_body,
        mesh=plsc.VectorSubcoreMesh(core_axis_name="c", subcore_axis_name="s"),
        out_type=[
            jax.ShapeDtypeStruct((2, num_tokens), jnp.float32),
            jax.ShapeDtypeStruct((2, num_tokens), jnp.int32),
            jax.ShapeDtypeStruct((_NW, _NUM_EXPERTS, _L), jnp.float32),
            jax.ShapeDtypeStruct((_NW, _NUM_EXPERTS, _L), jnp.float32),
        ],
        scratch_types=[
            pltpu.VMEM((_NUM_EXPERTS, chunk), jnp.float32),
            pltpu.VMEM((2, chunk), jnp.float32),
            pltpu.VMEM((2, chunk), jnp.int32),
            pltpu.VMEM((_NUM_EXPERTS, _L), jnp.float32),
            pltpu.VMEM((_NUM_EXPERTS, _L), jnp.float32),
            pltpu.SemaphoreType.DMA,
        ],
    )
    w2, e2, ps_part, cnt_part = sc_router(logits_t)

    tile2 = 8192
    weights, experts, loss = pl.pallas_call(
        functools.partial(_finish_body, num_tokens=num_tokens),
        grid=(num_tokens // tile2,),
        in_specs=[
            pl.BlockSpec((2, tile2), lambda i: (0, i)),
            pl.BlockSpec((2, tile2), lambda i: (0, i)),
            pl.BlockSpec((_NW, _NUM_EXPERTS, _L), lambda i: (0, 0, 0)),
            pl.BlockSpec((_NW, _NUM_EXPERTS, _L), lambda i: (0, 0, 0)),
        ],
        out_specs=[
            pl.BlockSpec((tile2, 2), lambda i: (i, 0)),
            pl.BlockSpec((tile2, 2), lambda i: (i, 0)),
            pl.BlockSpec((1, 1), lambda i: (0, 0)),
        ],
        out_shape=[
            jax.ShapeDtypeStruct((num_tokens, _TOP_K), jnp.float32),
            jax.ShapeDtypeStruct((num_tokens, _TOP_K), jnp.int32),
            jax.ShapeDtypeStruct((1, 1), jnp.float32),
        ],
    )(w2, e2, ps_part, cnt_part)

    return (weights.reshape(batch, seq, _TOP_K),
            experts.reshape(batch, seq, _TOP_K),
            loss[0, 0])


# speedup vs baseline: 1.5377x; 1.5377x over previous
import functools

import jax
import jax.numpy as jnp
from jax import lax
from jax.experimental import pallas as pl
from jax.experimental.pallas import tpu as pltpu
from jax.experimental.pallas import tpu_sc as plsc

_NUM_EXPERTS = 64
_TOP_K = 2
_ALPHA = 0.01
_TILE = 4096

_NC = 2
_NS = 16
_NW = _NC * _NS
_L = 16


def _matmul_body(x_ref, wt_ref, lt_out):
    tile = x_ref.shape[0]
    logits = jnp.dot(x_ref[...], wt_ref[...],
                     preferred_element_type=jnp.float32)
    sub = tile // 1024
    lt = logits.reshape(sub, 1024, _NUM_EXPERTS).transpose(0, 2, 1)
    lt_out[...] = lt


def _sc_router_body(lt_hbm, w_hbm, e_hbm, ps_hbm, cnt_hbm,
                    blk, wstage, estage, psacc, cntacc, dsem):
    wid = lax.axis_index("s") * _NC + lax.axis_index("c")
    chunk = blk.shape[1]
    n_groups = chunk // _L

    cp = pltpu.async_copy(lt_hbm.at[wid], blk, dsem)

    zf = jnp.zeros((_L,), jnp.float32)

    def zero_body(e, _):
        psacc[e, :] = zf
        cntacc[e, :] = zf
        return 0

    lax.fori_loop(0, _NUM_EXPERTS, zero_body, 0)
    cp.wait()

    def group_body(g, _):
        sl = pl.ds(g * _L, _L)

        m1 = blk[0, sl]
        i1 = jnp.zeros((_L,), jnp.int32)
        m2 = jnp.full((_L,), -jnp.inf, jnp.float32)
        i2 = jnp.zeros((_L,), jnp.int32)
        for e in range(1, _NUM_EXPERTS):
            v = blk[e, sl]
            ev = jnp.full((_L,), e, jnp.int32)
            gt1 = v > m1
            gt2 = v > m2
            m2 = jnp.where(gt1, m1, jnp.where(gt2, v, m2))
            i2 = jnp.where(gt1, i1, jnp.where(gt2, ev, i2))
            m1 = jnp.where(gt1, v, m1)
            i1 = jnp.where(gt1, ev, i1)

        z = jnp.zeros((_L,), jnp.float32)
        for e in range(_NUM_EXPERTS):
            ev = jnp.exp(blk[e, sl] - m1)
            blk[e, sl] = ev
            z = z + ev
        rz = 1.0 / z

        one = jnp.ones((_L,), jnp.float32)
        for e in range(_NUM_EXPERTS):
            hits = (jnp.where(i1 == e, one, 0.0) +
                    jnp.where(i2 == e, one, 0.0))
            psacc[e, :] = psacc[e, :] + blk[e, sl] * rz
            cntacc[e, :] = cntacc[e, :] + hits

        t = jnp.exp(m2 - m1)
        w1 = 1.0 / (1.0 + t)
        wstage[0, sl] = w1
        wstage[1, sl] = 1.0 - w1
        estage[0, sl] = i1
        estage[1, sl] = i2
        return 0

    lax.fori_loop(0, n_groups, group_body, 0)

    base = wid * chunk
    pltpu.sync_copy(wstage.at[0], w_hbm.at[0, pl.ds(base, chunk)])
    pltpu.sync_copy(wstage.at[1], w_hbm.at[1, pl.ds(base, chunk)])
    pltpu.sync_copy(estage.at[0], e_hbm.at[0, pl.ds(base, chunk)])
    pltpu.sync_copy(estage.at[1], e_hbm.at[1, pl.ds(base, chunk)])
    pltpu.sync_copy(psacc, ps_hbm.at[wid])
    pltpu.sync_copy(cntacc, cnt_hbm.at[wid])


def _finish_body(w2_ref, e2_ref, ps_ref, cnt_ref, w_out, e_out, loss_out, *,
                 num_tokens):
    w_out[...] = w2_ref[...].T
    e_out[...] = e2_ref[...].T

    @pl.when(pl.program_id(0) == 0)
    def _loss():
        psum = jnp.sum(ps_ref[...], axis=(0, 2))
        cnt = jnp.sum(cnt_ref[...], axis=(0, 2))
        scale = _ALPHA * _NUM_EXPERTS / (num_tokens * num_tokens)
        loss_out[...] = scale * jnp.sum(psum * cnt, keepdims=True)[None]


def kernel(hidden_states, W_gate):
    batch, seq, hidden = hidden_states.shape
    num_tokens = batch * seq
    x = hidden_states.reshape(num_tokens, hidden)
    wt = W_gate.T
    chunk = num_tokens // _NW
    n_steps = num_tokens // _TILE
    sub = _TILE // 1024

    logits_t = pl.pallas_call(
        _matmul_body,
        grid=(n_steps,),
        in_specs=[
            pl.BlockSpec((_TILE, hidden), lambda i: (i, 0)),
            pl.BlockSpec((hidden, _NUM_EXPERTS), lambda i: (0, 0)),
        ],
        out_specs=pl.BlockSpec((sub, _NUM_EXPERTS, 1024), lambda i: (i, 0, 0)),
        out_shape=jax.ShapeDtypeStruct((num_tokens // 1024, _NUM_EXPERTS,
                                        1024), jnp.float32),
    )(x, wt)

    sc_router = pl.kernel(
        _sc_router_body,
        mesh=plsc.VectorSubcoreMesh(core_axis_name="c", subcore_axis_name="s"),
        out_type=[
            jax.ShapeDtypeStruct((2, num_tokens), jnp.float32),
            jax.ShapeDtypeStruct((2, num_tokens), jnp.int32),
            jax.ShapeDtypeStruct((_NW, _NUM_EXPERTS, _L), jnp.float32),
            jax.ShapeDtypeStruct((_NW, _NUM_EXPERTS, _L), jnp.float32),
        ],
        scratch_types=[
            pltpu.VMEM((_NUM_EXPERTS, chunk), jnp.float32),
            pltpu.VMEM((2, chunk), jnp.float32),
            pltpu.VMEM((2, chunk), jnp.int32),
            pltpu.VMEM((_NUM_EXPERTS, _L), jnp.float32),
            pltpu.VMEM((_NUM_EXPERTS, _L), jnp.float32),
            pltpu.SemaphoreType.DMA,
        ],
    )
    w2, e2, ps_part, cnt_part = sc_router(logits_t)

    tile2 = 8192
    weights, experts, loss = pl.pallas_call(
        functools.partial(_finish_body, num_tokens=num_tokens),
        grid=(num_tokens // tile2,),
        in_specs=[
            pl.BlockSpec((2, tile2), lambda i: (0, i)),
            pl.BlockSpec((2, tile2), lambda i: (0, i)),
            pl.BlockSpec((_NW, _NUM_EXPERTS, _L), lambda i: (0, 0, 0)),
            pl.BlockSpec((_NW, _NUM_EXPERTS, _L), lambda i: (0, 0, 0)),
        ],
        out_specs=[
            pl.BlockSpec((tile2, 2), lambda i: (i, 0)),
            pl.BlockSpec((tile2, 2), lambda i: (i, 0)),
            pl.BlockSpec((1, 1), lambda i: (0, 0)),
        ],
        out_shape=[
            jax.ShapeDtypeStruct((num_tokens, _TOP_K), jnp.float32),
            jax.ShapeDtypeStruct((num_tokens, _TOP_K), jnp.int32),
            jax.ShapeDtypeStruct((1, 1), jnp.float32),
        ],
    )(w2, e2, ps_part, cnt_part)

    return (weights.reshape(batch, seq, _TOP_K),
            experts.reshape(batch, seq, _TOP_K),
            loss[0, 0])
